# baseline (speedup 1.0000x reference)
import functools

import jax
import jax.numpy as jnp
import numpy as np
from jax import lax
from jax.experimental import pallas as pl
from jax.experimental.pallas import tpu as pltpu
from jax.experimental.pallas import tpu_sc as plsc

_F = 26
_STRIDE = 38461
_B = 16384
_D = 16
_N = _B * _F
_NC, _NS, _L = 2, 16, 16
_NW = _NC * _NS
_PER_W = _N // _NW
_G = 128
_NG = _PER_W // _G
_C = 1664
_GPC = _C // _G
_NCHUNK = _PER_W // _C

_OFFS = np.tile(np.arange(_F, dtype=np.int32) * _STRIDE, _PER_W // _F)
_OFFS2D = _OFFS.reshape(_NG, _G)


@functools.partial(
    pl.kernel,
    out_type=jax.ShapeDtypeStruct((_N, _D), jnp.float32),
    mesh=plsc.VectorSubcoreMesh(core_axis_name="c", subcore_axis_name="s"),
    scratch_types=[
        pltpu.VMEM((_NG, _G), jnp.int32),
        pltpu.VMEM((_NG, _G), jnp.int32),
        pltpu.VMEM((_C, _D), jnp.float32),
        pltpu.SemaphoreType.DMA,
    ],
    compiler_params=pltpu.CompilerParams(use_tc_tiling_on_sc=False),
)
def _sc_embedding_gather(table_h, x_h, offs_h, out_h, idx_v, offs_v, rows_v, sem):
    wid = lax.axis_index("s") * _NC + lax.axis_index("c")
    row_base = wid * _NG
    out_base = wid * _PER_W

    pltpu.sync_copy(x_h.at[pl.ds(row_base, _NG)], idx_v)
    pltpu.sync_copy(offs_h, offs_v)

    @pl.loop(0, _NG)
    def _(g):
        @pl.loop(0, _G // _L)
        def _(l):
            s = pl.ds(l * _L, _L)
            idx_v[g, s] = idx_v[g, s] + offs_v[g, s]

    for c in range(_NCHUNK):
        descs = []
        for j in range(_GPC):
            g = c * _GPC + j
            descs.append(
                pltpu.async_copy(
                    table_h.at[idx_v.at[g]],
                    rows_v.at[pl.ds(j * _G, _G)],
                    sem,
                )
            )
        for d in descs:
            d.wait()
        pltpu.sync_copy(rows_v, out_h.at[pl.ds(out_base + c * _C, _C)])


def kernel(x, weights_embed):
    x_flat = x.reshape(_N // _G, _G)
    offs = jnp.asarray(_OFFS2D)
    out = _sc_embedding_gather(weights_embed, x_flat, offs)
    return out.reshape(_B, _F, _D)


# speedup vs baseline: 1.0307x; 1.0307x over previous
import functools

import jax
import jax.numpy as jnp
from jax import lax
from jax.experimental import pallas as pl
from jax.experimental.pallas import tpu as pltpu
from jax.experimental.pallas import tpu_sc as plsc

_F = 26
_STRIDE = 38461
_B = 16384
_D = 16
_V = _F * _STRIDE + 1
_VP = 1000064
_Q = _VP // 8
_NC, _NS, _L = 2, 16, 16
_NW = _NC * _NS
_BPW = _B // _NW
_BT = _BPW // 128
_NK = _F * _BT

_MESH = dict(
    mesh=plsc.VectorSubcoreMesh(core_axis_name="c", subcore_axis_name="s"),
    compiler_params=pltpu.CompilerParams(
        use_tc_tiling_on_sc=True, needs_layout_passes=False
    ),
)

_FULL_CT = _VP // 128 - 1
_CPW = _FULL_CT // _NW
_CXT = _FULL_CT - _CPW * _NW
_TAIL = _V - _FULL_CT * 128


@functools.partial(
    pl.kernel,
    out_type=jax.ShapeDtypeStruct((_Q, 128), jnp.float32),
    scratch_types=[
        pltpu.VMEM((_D, 128), jnp.float32),
        pltpu.VMEM((_D, 128), jnp.float32),
    ],
    **_MESH,
)
def _table_convert(tbt_h, tail_h, out_h, vbuf, trows):
    wid = lax.axis_index("s") * _NC + lax.axis_index("c")
    base = wid * _CPW + jnp.minimum(wid, _CXT)
    n = _CPW + jnp.where(wid < _CXT, 1, 0)

    @pl.loop(0, n)
    def _(i):
        c = base + i
        pltpu.sync_copy(tbt_h.at[pl.ds(0, 8), pl.ds(c * 128, 128)], vbuf.at[pl.ds(0, 8)])
        pltpu.sync_copy(tbt_h.at[pl.ds(8, 8), pl.ds(c * 128, 128)], vbuf.at[pl.ds(8, 8)])

        @pl.loop(0, 128)
        def _(l):
            vec = plsc.load_gather(
                vbuf, [lax.iota(jnp.int32, _L), jnp.full((_L,), l, jnp.int32)]
            )
            trows[lax.shift_right_logical(l, 3),
                  pl.ds(lax.rem(l, 8) * _D, _D)] = vec

        pltpu.sync_copy(trows, out_h.at[pl.ds(c * 16, 16)])

    @pl.when(wid == _NW - 1)
    def _():
        pltpu.sync_copy(tail_h, trows)
        pltpu.sync_copy(trows, out_h.at[pl.ds(_FULL_CT * 16, 16)])


@functools.partial(
    pl.kernel,
    out_type=jax.ShapeDtypeStruct((_F, _D, _B), jnp.float32),
    scratch_types=[
        pltpu.VMEM((_F, _BPW), jnp.int32),
        pltpu.VMEM((2, 128), jnp.int32),
        pltpu.VMEM((2, 128), jnp.int32),
        pltpu.VMEM((128, 128), jnp.float32),
        pltpu.VMEM((128, 128), jnp.float32),
        pltpu.VMEM((_D, 128), jnp.float32),
        pltpu.SemaphoreType.DMA,
    ],
    **_MESH,
)
def _lookup(xt_h, tbl_h, out_h, xv, idxb, idxq, blk0, blk1, tbuf, gsem):
    wid = lax.axis_index("s") * _NC + lax.axis_index("c")
    b0 = wid * _BPW

    pltpu.sync_copy(xt_h.at[pl.ds(0, 8), pl.ds(b0, _BPW)], xv.at[pl.ds(0, 8)])
    pltpu.sync_copy(xt_h.at[pl.ds(8, 8), pl.ds(b0, _BPW)], xv.at[pl.ds(8, 8)])
    pltpu.sync_copy(xt_h.at[pl.ds(16, 8), pl.ds(b0, _BPW)], xv.at[pl.ds(16, 8)])
    pltpu.sync_copy(xt_h.at[pl.ds(24, 2), pl.ds(b0, _BPW)], xv.at[pl.ds(24, 2)])

    blks = (blk0, blk1)

    def build_and_issue(k, slot):
        f = lax.shift_right_logical(k, 2)
        bt = lax.rem(k, 4)
        for j in range(128 // _L):
            s = pl.ds(bt * 128 + j * _L, _L)
            idx = xv[f, s] + f * _STRIDE
            idxb[slot, pl.ds(j * _L, _L)] = idx
            idxq[slot, pl.ds(j * _L, _L)] = lax.shift_right_logical(idx, 3)
        return pltpu.async_copy(tbl_h.at[idxq.at[slot]], blks[slot], gsem)

    def extract_and_store(k, slot, desc):
        f = lax.shift_right_logical(k, 2)
        bt = lax.rem(k, 4)
        desc.wait()

        @pl.loop(0, 128 // _L)
        def _(t):
            bl_vec = t * _L + lax.iota(jnp.int32, _L)
            col0 = lax.rem(idxb[slot, pl.ds(t * _L, _L)], 8) * _D
            for d in range(_D):
                vec = plsc.load_gather(blks[slot], [bl_vec, col0 + d])
                tbuf[d, pl.ds(t * _L, _L)] = vec

        pltpu.sync_copy(tbuf, out_h.at[f, :, pl.ds(b0 + bt * 128, 128)])

    @pl.loop(0, _NK, step=2)
    def _(k):
        d0 = build_and_issue(k, 0)
        d1 = build_and_issue(k + 1, 1)
        extract_and_store(k, 0, d0)
        extract_and_store(k + 1, 1, d1)


def kernel(x, weights_embed):
    tailp = jnp.pad(weights_embed[_FULL_CT * 128:], ((0, 128 - _TAIL), (0, 0)))
    tail = tailp.reshape(_D, 128)
    tbl = _table_convert(weights_embed.T, tail)
    out3 = _lookup(x.T, tbl)
    return jnp.transpose(out3, (2, 0, 1))


# speedup vs baseline: 1.5201x; 1.4748x over previous
import functools

import jax
import jax.numpy as jnp
from jax import lax
from jax.experimental import pallas as pl
from jax.experimental.pallas import tpu as pltpu
from jax.experimental.pallas import tpu_sc as plsc

_F = 26
_STRIDE = 38461
_B = 16384
_D = 16
_V = _F * _STRIDE + 1
_VP = 1000064
_Q = _VP // 8
_NC, _NS, _L = 2, 16, 16
_NW = _NC * _NS
_BPW = _B // _NW
_BT = _BPW // 128

_MESH = dict(
    mesh=plsc.VectorSubcoreMesh(core_axis_name="c", subcore_axis_name="s"),
    compiler_params=pltpu.CompilerParams(
        use_tc_tiling_on_sc=True, needs_layout_passes=False
    ),
)

_FULL_CT = _VP // 128 - 1
_TAIL = _V - _FULL_CT * 128
_GPW = 61


@functools.partial(
    pl.kernel,
    out_type=jax.ShapeDtypeStruct((_Q, 128), jnp.float32),
    scratch_types=[
        pltpu.VMEM((2, _D, 512), jnp.float32),
        pltpu.VMEM((2, 64, 128), jnp.float32),
        pltpu.SemaphoreType.DMA,
        pltpu.SemaphoreType.DMA,
    ],
    **_MESH,
)
def _table_convert(tbt_h, tail_h, out_h, vbuf, trows, lsem, ssem):
    wid = lax.axis_index("s") * _NC + lax.axis_index("c")
    base = wid * _GPW

    def load(i, slot):
        c0 = (base + i) * 512
        d0 = pltpu.async_copy(
            tbt_h.at[pl.ds(0, 8), pl.ds(c0, 512)], vbuf.at[slot, pl.ds(0, 8)], lsem)
        d1 = pltpu.async_copy(
            tbt_h.at[pl.ds(8, 8), pl.ds(c0, 512)], vbuf.at[slot, pl.ds(8, 8)], lsem)
        return (d0, d1)

    def transpose(slot):
        @pl.loop(0, 512, unroll=8)
        def _(l):
            vec = plsc.load_gather(
                vbuf.at[slot], [lax.iota(jnp.int32, _L), jnp.full((_L,), l, jnp.int32)]
            )
            trows[slot, lax.shift_right_logical(l, 3),
                  pl.ds(lax.rem(l, 8) * _D, _D)] = vec

    def store(i, slot):
        return pltpu.async_copy(
            trows.at[slot], out_h.at[pl.ds((base + i) * 64, 64)], ssem)

    @pl.loop(0, _GPW - 1, step=2)
    def _(i):
        l0 = load(i, 0)
        l1 = load(i + 1, 1)
        for d in l0:
            d.wait()
        transpose(0)
        s0 = store(i, 0)
        for d in l1:
            d.wait()
        transpose(1)
        s1 = store(i + 1, 1)
        s0.wait()
        s1.wait()

    for d in load(_GPW - 1, 0):
        d.wait()
    transpose(0)
    pltpu.sync_copy(trows.at[0], out_h.at[pl.ds((base + _GPW - 1) * 64, 64)])

    @pl.when(wid == _NW - 1)
    def _():
        c0 = 1952 * 512
        pltpu.sync_copy(tbt_h.at[pl.ds(0, 8), pl.ds(c0, 512)], vbuf.at[0, pl.ds(0, 8)])
        pltpu.sync_copy(tbt_h.at[pl.ds(8, 8), pl.ds(c0, 512)], vbuf.at[0, pl.ds(8, 8)])
        transpose(0)
        pltpu.sync_copy(trows.at[0], out_h.at[pl.ds(1952 * 64, 64)])
        pltpu.sync_copy(tail_h, trows.at[0, pl.ds(0, 16)])
        pltpu.sync_copy(trows.at[0, pl.ds(0, 16)], out_h.at[pl.ds(_FULL_CT * 16, 16)])


@functools.partial(
    pl.kernel,
    out_type=jax.ShapeDtypeStruct((_F, _D, _B), jnp.float32),
    scratch_types=[
        pltpu.VMEM((_F, _BPW), jnp.int32),
        pltpu.VMEM((2, 128), jnp.int32),
        pltpu.VMEM((2, 128), jnp.int32),
        pltpu.VMEM((128, 128), jnp.float32),
        pltpu.VMEM((128, 128), jnp.float32),
        pltpu.VMEM((_D, _BPW), jnp.float32),
        pltpu.SemaphoreType.DMA,
    ],
    **_MESH,
)
def _lookup(xt_h, tbl_h, out_h, xv, idxb, idxq, blk0, blk1, tbuf, gsem):
    wid = lax.axis_index("s") * _NC + lax.axis_index("c")
    b0 = wid * _BPW

    pltpu.sync_copy(xt_h.at[pl.ds(0, 8), pl.ds(b0, _BPW)], xv.at[pl.ds(0, 8)])
    pltpu.sync_copy(xt_h.at[pl.ds(8, 8), pl.ds(b0, _BPW)], xv.at[pl.ds(8, 8)])
    pltpu.sync_copy(xt_h.at[pl.ds(16, 8), pl.ds(b0, _BPW)], xv.at[pl.ds(16, 8)])
    pltpu.sync_copy(xt_h.at[pl.ds(24, 2), pl.ds(b0, _BPW)], xv.at[pl.ds(24, 2)])

    blks = (blk0, blk1)

    def build_and_issue(f, bt, slot):
        for j in range(128 // _L):
            s = pl.ds(bt * 128 + j * _L, _L)
            idx = xv[f, s] + f * _STRIDE
            idxb[slot, pl.ds(j * _L, _L)] = idx
            idxq[slot, pl.ds(j * _L, _L)] = lax.shift_right_logical(idx, 3)
        return pltpu.async_copy(tbl_h.at[idxq.at[slot]], blks[slot], gsem)

    def extract(bt, slot, desc):
        desc.wait()

        @pl.loop(0, 128 // _L)
        def _(t):
            bl_vec = t * _L + lax.iota(jnp.int32, _L)
            col0 = lax.rem(idxb[slot, pl.ds(t * _L, _L)], 8) * _D
            for d in range(_D):
                vec = plsc.load_gather(blks[slot], [bl_vec, col0 + d])
                tbuf[d, pl.ds(bt * 128 + t * _L, _L)] = vec

    @pl.loop(0, _F)
    def _(f):
        descs = [None, None]
        descs[0] = build_and_issue(f, 0, 0)
        for bt in range(_BT):
            if bt + 1 < _BT:
                descs[(bt + 1) % 2] = build_and_issue(f, bt + 1, (bt + 1) % 2)
            extract(bt, bt % 2, descs[bt % 2])
        pltpu.sync_copy(tbuf, out_h.at[f, :, pl.ds(b0, _BPW)])


def kernel(x, weights_embed):
    tailp = jnp.pad(weights_embed[_FULL_CT * 128:], ((0, 128 - _TAIL), (0, 0)))
    tail = tailp.reshape(_D, 128)
    tbl = _table_convert(weights_embed.T, tail)
    out3 = _lookup(x.T, tbl)
    return jnp.transpose(out3, (2, 0, 1))


# speedup vs baseline: 1.5218x; 1.0011x over previous
import functools

import jax
import jax.numpy as jnp
from jax import lax
from jax.experimental import pallas as pl
from jax.experimental.pallas import tpu as pltpu
from jax.experimental.pallas import tpu_sc as plsc

_F = 26
_STRIDE = 38461
_B = 16384
_D = 16
_V = _F * _STRIDE + 1
_VP = 1000064
_Q = _VP // 8
_NC, _NS, _L = 2, 16, 16
_NW = _NC * _NS
_BPW = _B // _NW
_BT = _BPW // 128

_MESH = dict(
    mesh=plsc.VectorSubcoreMesh(core_axis_name="c", subcore_axis_name="s"),
    compiler_params=pltpu.CompilerParams(
        use_tc_tiling_on_sc=True, needs_layout_passes=False
    ),
)

_FULL_CT = _VP // 128 - 1
_TAIL = _V - _FULL_CT * 128
_GPW = 61


@functools.partial(
    pl.kernel,
    out_type=jax.ShapeDtypeStruct((_Q, 128), jnp.float32),
    scratch_types=[
        pltpu.VMEM((2, _D, 512), jnp.float32),
        pltpu.VMEM((2, 64, 128), jnp.float32),
        pltpu.SemaphoreType.DMA,
        pltpu.SemaphoreType.DMA,
    ],
    **_MESH,
)
def _table_convert(tbt_h, tail_h, out_h, vbuf, trows, lsem, ssem):
    wid = lax.axis_index("s") * _NC + lax.axis_index("c")
    base = wid * _GPW

    def load(i, slot):
        c0 = (base + i) * 512
        d0 = pltpu.async_copy(
            tbt_h.at[pl.ds(0, 8), pl.ds(c0, 512)], vbuf.at[slot, pl.ds(0, 8)], lsem)
        d1 = pltpu.async_copy(
            tbt_h.at[pl.ds(8, 8), pl.ds(c0, 512)], vbuf.at[slot, pl.ds(8, 8)], lsem)
        return (d0, d1)

    def transpose(slot):
        @pl.loop(0, 64)
        def _(r):
            for j in range(8):
                l = r * 8 + j
                vec = plsc.load_gather(
                    vbuf.at[slot],
                    [lax.iota(jnp.int32, _L), jnp.full((_L,), l, jnp.int32)],
                )
                trows[slot, r, pl.ds(j * _D, _D)] = vec

    def store(i, slot):
        return pltpu.async_copy(
            trows.at[slot], out_h.at[pl.ds((base + i) * 64, 64)], ssem)

    @pl.loop(0, _GPW - 1, step=2)
    def _(i):
        l0 = load(i, 0)
        l1 = load(i + 1, 1)
        for d in l0:
            d.wait()
        transpose(0)
        s0 = store(i, 0)
        for d in l1:
            d.wait()
        transpose(1)
        s1 = store(i + 1, 1)
        s0.wait()
        s1.wait()

    for d in load(_GPW - 1, 0):
        d.wait()
    transpose(0)
    pltpu.sync_copy(trows.at[0], out_h.at[pl.ds((base + _GPW - 1) * 64, 64)])

    @pl.when(wid == _NW - 1)
    def _():
        c0 = 1952 * 512
        pltpu.sync_copy(tbt_h.at[pl.ds(0, 8), pl.ds(c0, 512)], vbuf.at[0, pl.ds(0, 8)])
        pltpu.sync_copy(tbt_h.at[pl.ds(8, 8), pl.ds(c0, 512)], vbuf.at[0, pl.ds(8, 8)])
        transpose(0)
        pltpu.sync_copy(trows.at[0], out_h.at[pl.ds(1952 * 64, 64)])
        pltpu.sync_copy(tail_h, trows.at[0, pl.ds(0, 16)])
        pltpu.sync_copy(trows.at[0, pl.ds(0, 16)], out_h.at[pl.ds(_FULL_CT * 16, 16)])


@functools.partial(
    pl.kernel,
    out_type=jax.ShapeDtypeStruct((_F, _D, _B), jnp.float32),
    scratch_types=[
        pltpu.VMEM((_F, _BPW), jnp.int32),
        pltpu.VMEM((2, 128), jnp.int32),
        pltpu.VMEM((2, 128), jnp.int32),
        pltpu.VMEM((128, 128), jnp.float32),
        pltpu.VMEM((128, 128), jnp.float32),
        pltpu.VMEM((_D, _BPW), jnp.float32),
        pltpu.SemaphoreType.DMA,
    ],
    **_MESH,
)
def _lookup(xt_h, tbl_h, out_h, xv, idxb, idxq, blk0, blk1, tbuf, gsem):
    wid = lax.axis_index("s") * _NC + lax.axis_index("c")
    b0 = wid * _BPW

    pltpu.sync_copy(xt_h.at[pl.ds(0, 8), pl.ds(b0, _BPW)], xv.at[pl.ds(0, 8)])
    pltpu.sync_copy(xt_h.at[pl.ds(8, 8), pl.ds(b0, _BPW)], xv.at[pl.ds(8, 8)])
    pltpu.sync_copy(xt_h.at[pl.ds(16, 8), pl.ds(b0, _BPW)], xv.at[pl.ds(16, 8)])
    pltpu.sync_copy(xt_h.at[pl.ds(24, 2), pl.ds(b0, _BPW)], xv.at[pl.ds(24, 2)])

    blks = (blk0, blk1)

    def build_and_issue(f, bt, slot):
        for j in range(128 // _L):
            s = pl.ds(bt * 128 + j * _L, _L)
            idx = xv[f, s] + f * _STRIDE
            idxb[slot, pl.ds(j * _L, _L)] = idx
            idxq[slot, pl.ds(j * _L, _L)] = lax.shift_right_logical(idx, 3)
        return pltpu.async_copy(tbl_h.at[idxq.at[slot]], blks[slot], gsem)

    def extract(bt, slot, desc):
        desc.wait()

        @pl.loop(0, 128 // _L)
        def _(t):
            bl_vec = t * _L + lax.iota(jnp.int32, _L)
            col0 = lax.rem(idxb[slot, pl.ds(t * _L, _L)], 8) * _D
            for d in range(_D):
                vec = plsc.load_gather(blks[slot], [bl_vec, col0 + d])
                tbuf[d, pl.ds(bt * 128 + t * _L, _L)] = vec

    @pl.loop(0, _F)
    def _(f):
        descs = [None, None]
        descs[0] = build_and_issue(f, 0, 0)
        for bt in range(_BT):
            if bt + 1 < _BT:
                descs[(bt + 1) % 2] = build_and_issue(f, bt + 1, (bt + 1) % 2)
            extract(bt, bt % 2, descs[bt % 2])
        pltpu.sync_copy(tbuf, out_h.at[f, :, pl.ds(b0, _BPW)])


def kernel(x, weights_embed):
    tailp = jnp.pad(weights_embed[_FULL_CT * 128:], ((0, 128 - _TAIL), (0, 0)))
    tail = tailp.reshape(_D, 128)
    tbl = _table_convert(weights_embed.T, tail)
    out3 = _lookup(x.T, tbl)
    return jnp.transpose(out3, (2, 0, 1))


# speedup vs baseline: 2.6304x; 1.7285x over previous
import functools

import jax
import jax.numpy as jnp
from jax import lax
from jax.experimental import pallas as pl
from jax.experimental.pallas import tpu as pltpu
from jax.experimental.pallas import tpu_sc as plsc

_F = 26
_STRIDE = 38461
_B = 16384
_D = 16
_V = _F * _STRIDE + 1
_VP = 1000064
_Q = _VP // 8
_NC, _NS, _L = 2, 16, 16
_NW = _NC * _NS
_BPW = _B // _NW
_BT = _BPW // 128

_MESH = dict(
    mesh=plsc.VectorSubcoreMesh(core_axis_name="c", subcore_axis_name="s"),
    compiler_params=pltpu.CompilerParams(
        use_tc_tiling_on_sc=True, needs_layout_passes=False
    ),
)

_FULL_CT = _VP // 128 - 1
_TAIL = _V - _FULL_CT * 128
_GPW = 61


@functools.partial(
    pl.kernel,
    out_type=jax.ShapeDtypeStruct((_Q, 128), jnp.float32),
    scratch_types=[
        pltpu.VMEM((2, _D, 512), jnp.float32),
        pltpu.VMEM((2, 64, 128), jnp.float32),
        pltpu.SemaphoreType.DMA,
        pltpu.SemaphoreType.DMA,
    ],
    **_MESH,
)
def _table_convert(tbt_h, tail_h, out_h, vbuf, trows, lsem, ssem):
    wid = lax.axis_index("s") * _NC + lax.axis_index("c")
    base = wid * _GPW

    def load(i, slot):
        c0 = (base + i) * 512
        d0 = pltpu.async_copy(
            tbt_h.at[pl.ds(0, 8), pl.ds(c0, 512)], vbuf.at[slot, pl.ds(0, 8)], lsem)
        d1 = pltpu.async_copy(
            tbt_h.at[pl.ds(8, 8), pl.ds(c0, 512)], vbuf.at[slot, pl.ds(8, 8)], lsem)
        return (d0, d1)

    def transpose(slot):
        rows0 = lax.shift_right_logical(lax.iota(jnp.int32, _L), 3)
        lane0 = lax.rem(lax.iota(jnp.int32, _L), 8) * _D

        @pl.loop(0, 32)
        def _(g):
            rows_i = 2 * g + rows0
            for d in range(_D):
                vec = vbuf[slot, d, pl.ds(g * _L, _L)]
                plsc.store_scatter(trows.at[slot], [rows_i, lane0 + d], vec)

    def store(i, slot):
        return pltpu.async_copy(
            trows.at[slot], out_h.at[pl.ds((base + i) * 64, 64)], ssem)

    @pl.loop(0, _GPW - 1, step=2)
    def _(i):
        l0 = load(i, 0)
        l1 = load(i + 1, 1)
        for d in l0:
            d.wait()
        transpose(0)
        s0 = store(i, 0)
        for d in l1:
            d.wait()
        transpose(1)
        s1 = store(i + 1, 1)
        s0.wait()
        s1.wait()

    for d in load(_GPW - 1, 0):
        d.wait()
    transpose(0)
    pltpu.sync_copy(trows.at[0], out_h.at[pl.ds((base + _GPW - 1) * 64, 64)])

    @pl.when(wid == _NW - 1)
    def _():
        c0 = 1952 * 512
        pltpu.sync_copy(tbt_h.at[pl.ds(0, 8), pl.ds(c0, 512)], vbuf.at[0, pl.ds(0, 8)])
        pltpu.sync_copy(tbt_h.at[pl.ds(8, 8), pl.ds(c0, 512)], vbuf.at[0, pl.ds(8, 8)])
        transpose(0)
        pltpu.sync_copy(trows.at[0], out_h.at[pl.ds(1952 * 64, 64)])
        pltpu.sync_copy(tail_h, trows.at[0, pl.ds(0, 16)])
        pltpu.sync_copy(trows.at[0, pl.ds(0, 16)], out_h.at[pl.ds(_FULL_CT * 16, 16)])


@functools.partial(
    pl.kernel,
    out_type=jax.ShapeDtypeStruct((_F, _D, _B), jnp.float32),
    scratch_types=[
        pltpu.VMEM((_F, _BPW), jnp.int32),
        pltpu.VMEM((2, 128), jnp.int32),
        pltpu.VMEM((2, 128), jnp.int32),
        pltpu.VMEM((128, 128), jnp.float32),
        pltpu.VMEM((128, 128), jnp.float32),
        pltpu.VMEM((_D, _BPW), jnp.float32),
        pltpu.SemaphoreType.DMA,
    ],
    **_MESH,
)
def _lookup(xt_h, tbl_h, out_h, xv, idxb, idxq, blk0, blk1, tbuf, gsem):
    wid = lax.axis_index("s") * _NC + lax.axis_index("c")
    b0 = wid * _BPW

    pltpu.sync_copy(xt_h.at[pl.ds(0, 8), pl.ds(b0, _BPW)], xv.at[pl.ds(0, 8)])
    pltpu.sync_copy(xt_h.at[pl.ds(8, 8), pl.ds(b0, _BPW)], xv.at[pl.ds(8, 8)])
    pltpu.sync_copy(xt_h.at[pl.ds(16, 8), pl.ds(b0, _BPW)], xv.at[pl.ds(16, 8)])
    pltpu.sync_copy(xt_h.at[pl.ds(24, 2), pl.ds(b0, _BPW)], xv.at[pl.ds(24, 2)])

    blks = (blk0, blk1)

    def build_and_issue(f, bt, slot):
        for j in range(128 // _L):
            s = pl.ds(bt * 128 + j * _L, _L)
            idx = xv[f, s] + f * _STRIDE
            idxb[slot, pl.ds(j * _L, _L)] = idx
            idxq[slot, pl.ds(j * _L, _L)] = lax.shift_right_logical(idx, 3)
        return pltpu.async_copy(tbl_h.at[idxq.at[slot]], blks[slot], gsem)

    def extract(bt, slot, desc):
        desc.wait()

        @pl.loop(0, 128 // _L)
        def _(t):
            bl_vec = t * _L + lax.iota(jnp.int32, _L)
            col0 = lax.rem(idxb[slot, pl.ds(t * _L, _L)], 8) * _D
            for d in range(_D):
                vec = plsc.load_gather(blks[slot], [bl_vec, col0 + d])
                tbuf[d, pl.ds(bt * 128 + t * _L, _L)] = vec

    @pl.loop(0, _F)
    def _(f):
        descs = [None, None]
        descs[0] = build_and_issue(f, 0, 0)
        for bt in range(_BT):
            if bt + 1 < _BT:
                descs[(bt + 1) % 2] = build_and_issue(f, bt + 1, (bt + 1) % 2)
            extract(bt, bt % 2, descs[bt % 2])
        pltpu.sync_copy(tbuf, out_h.at[f, :, pl.ds(b0, _BPW)])


def kernel(x, weights_embed):
    tailp = jnp.pad(weights_embed[_FULL_CT * 128:], ((0, 128 - _TAIL), (0, 0)))
    tail = tailp.reshape(_D, 128)
    tbl = _table_convert(weights_embed.T, tail)
    out3 = _lookup(x.T, tbl)
    return jnp.transpose(out3, (2, 0, 1))
